# Initial kernel scaffold; baseline (speedup 1.0000x reference)
import jax
import jax.numpy as jnp
from jax.experimental import pallas as pl


def kernel(S, T, cur, prev, W_ad, b_ad, Wr1, br1, Wr2, br2):
    raise NotImplementedError("write your pallas kernel here")


# speedup vs baseline: 2.1955x; 2.1955x over previous
import jax
import jax.numpy as jnp
from jax.experimental import pallas as pl
from jax.experimental.pallas import tpu as pltpu

_B, _SW, _C, _H, _W = 8, 8, 192, 24, 24
_P = _H * _W
_RH = 64

_TEMP = 0.7
_ALPHA_ADV, _BETA_BAND = 1.0, 0.5
_BAND_L, _BAND_H = 0.05, 0.2
_W_HEUR, _W_LEAR = 0.5, 0.5

_PREC = jax.lax.Precision.HIGHEST


def _gater_body(cur_ref, prev_ref, S_ref, T_ref, W_ad_ref, b_ad_ref,
                U_T_ref, M_S_ref, c0_ref, Wr2_ref, br2_ref,
                y_ref, Tz_ref):
    hid = jnp.dot(M_S_ref[...], S_ref[0], precision=_PREC,
                  preferred_element_type=jnp.float32)
    hid = hid + c0_ref[...]
    for s in range(_SW):
        Tz_s = jnp.dot(W_ad_ref[s], T_ref[0, s], precision=_PREC,
                       preferred_element_type=jnp.float32)
        Tz_s = Tz_s + b_ad_ref[s]
        Tz_ref[s] = Tz_s
        hid = hid + jnp.dot(U_T_ref[s], Tz_s, precision=_PREC,
                            preferred_element_type=jnp.float32)

    h = jnp.maximum(hid, 0.0)
    learned = jnp.dot(Wr2_ref[...], h, precision=_PREC,
                      preferred_element_type=jnp.float32) + br2_ref[...]

    cur_c = cur_ref[0]
    prev_c = prev_ref[0]
    impr = prev_c - cur_c
    adv = impr - jnp.mean(impr, axis=0, keepdims=True)
    below = jnp.maximum(_BAND_L - cur_c, 0.0)
    above = jnp.maximum(cur_c - _BAND_H, 0.0)
    band = -(below * below + above * above)
    heur = _ALPHA_ADV * adv + _BETA_BAND * band

    logits = (_W_HEUR * heur + _W_LEAR * learned) / _TEMP
    m = jnp.max(logits, axis=0, keepdims=True)
    e = jnp.exp(logits - m)
    g = e / jnp.sum(e, axis=0, keepdims=True)

    acc = g[0:1, :] * Tz_ref[0]
    for s in range(1, _SW):
        acc = acc + g[s:s + 1, :] * Tz_ref[s]
    y_ref[0] = acc


def kernel(S, T, cur, prev, W_ad, b_ad, Wr1, br1, Wr2, br2):
    S3 = S.reshape(_B, _C, _P)
    T4 = T.reshape(_B, _SW, _C, _P)
    cur2 = cur.reshape(_B, _SW, 1)
    prev2 = prev.reshape(_B, _SW, 1)

    Wr1r = Wr1.reshape(_RH, _SW, 3, _C)
    A_S, A_T, A_D = Wr1r[:, :, 0], Wr1r[:, :, 1], Wr1r[:, :, 2]
    WS = A_S - A_D
    U_T = jnp.transpose(A_T + A_D, (1, 0, 2))
    M_S = jnp.einsum('rso,soc->rc', WS, W_ad, precision=_PREC)
    c0 = (jnp.einsum('rso,so->r', WS, b_ad, precision=_PREC)
          + br1).reshape(_RH, 1)
    b_ad3 = b_ad.reshape(_SW, _C, 1)
    br2c = br2.reshape(_SW, 1)

    full = lambda shape: pl.BlockSpec(shape, lambda b: (0,) * len(shape))
    y = pl.pallas_call(
        _gater_body,
        grid=(_B,),
        in_specs=[
            pl.BlockSpec((1, _SW, 1), lambda b: (b, 0, 0)),
            pl.BlockSpec((1, _SW, 1), lambda b: (b, 0, 0)),
            pl.BlockSpec((1, _C, _P), lambda b: (b, 0, 0)),
            pl.BlockSpec((1, _SW, _C, _P), lambda b: (b, 0, 0, 0)),
            full((_SW, _C, _C)),
            full((_SW, _C, 1)),
            full((_SW, _RH, _C)),
            full((_RH, _C)),
            full((_RH, 1)),
            full((_SW, _RH)),
            full((_SW, 1)),
        ],
        out_specs=pl.BlockSpec((1, _C, _P), lambda b: (b, 0, 0)),
        out_shape=jax.ShapeDtypeStruct((_B, _C, _P), jnp.float32),
        scratch_shapes=[pltpu.VMEM((_SW, _C, _P), jnp.float32)],
    )(cur2, prev2, S3, T4, W_ad, b_ad3, U_T, M_S, c0, Wr2, br2c)

    return y.reshape(_B, _C, _H, _W)


# speedup vs baseline: 3.1447x; 1.4323x over previous
import jax
import jax.numpy as jnp
from jax.experimental import pallas as pl
from jax.experimental.pallas import tpu as pltpu

_B, _SW, _C, _H, _W = 8, 8, 192, 24, 24
_P = _H * _W
_RH = 64

_TEMP = 0.7
_ALPHA_ADV, _BETA_BAND = 1.0, 0.5
_BAND_L, _BAND_H = 0.05, 0.2
_W_HEUR, _W_LEAR = 0.5, 0.5

_PREC = jax.lax.Precision.HIGHEST
_KPREC = jax.lax.Precision.DEFAULT


def _gater_body(cur_ref, prev_ref, S_ref, T_ref, W_ad_ref, b_ad_ref,
                U_T_ref, M_S_ref, c0_ref, Wr2_ref, br2_ref,
                y_ref, Tz_ref):
    hid = jnp.dot(M_S_ref[...], S_ref[0], precision=_KPREC,
                  preferred_element_type=jnp.float32)
    hid = hid + c0_ref[...]
    for s in range(_SW):
        Tz_s = jnp.dot(W_ad_ref[s], T_ref[0, s], precision=_KPREC,
                       preferred_element_type=jnp.float32)
        Tz_s = Tz_s + b_ad_ref[s]
        Tz_ref[s] = Tz_s
        hid = hid + jnp.dot(U_T_ref[s], Tz_s, precision=_KPREC,
                            preferred_element_type=jnp.float32)

    h = jnp.maximum(hid, 0.0)
    learned = jnp.dot(Wr2_ref[...], h, precision=_KPREC,
                      preferred_element_type=jnp.float32) + br2_ref[...]

    cur_c = cur_ref[0]
    prev_c = prev_ref[0]
    impr = prev_c - cur_c
    adv = impr - jnp.mean(impr, axis=0, keepdims=True)
    below = jnp.maximum(_BAND_L - cur_c, 0.0)
    above = jnp.maximum(cur_c - _BAND_H, 0.0)
    band = -(below * below + above * above)
    heur = _ALPHA_ADV * adv + _BETA_BAND * band

    logits = (_W_HEUR * heur + _W_LEAR * learned) / _TEMP
    m = jnp.max(logits, axis=0, keepdims=True)
    e = jnp.exp(logits - m)
    g = e / jnp.sum(e, axis=0, keepdims=True)

    acc = g[0:1, :] * Tz_ref[0]
    for s in range(1, _SW):
        acc = acc + g[s:s + 1, :] * Tz_ref[s]
    y_ref[0] = acc


def kernel(S, T, cur, prev, W_ad, b_ad, Wr1, br1, Wr2, br2):
    S3 = S.reshape(_B, _C, _P)
    T4 = T.reshape(_B, _SW, _C, _P)
    cur2 = cur.reshape(_B, _SW, 1)
    prev2 = prev.reshape(_B, _SW, 1)

    Wr1r = Wr1.reshape(_RH, _SW, 3, _C)
    A_S, A_T, A_D = Wr1r[:, :, 0], Wr1r[:, :, 1], Wr1r[:, :, 2]
    WS = A_S - A_D
    U_T = jnp.transpose(A_T + A_D, (1, 0, 2))
    M_S = jnp.einsum('rso,soc->rc', WS, W_ad, precision=_PREC)
    c0 = (jnp.einsum('rso,so->r', WS, b_ad, precision=_PREC)
          + br1).reshape(_RH, 1)
    b_ad3 = b_ad.reshape(_SW, _C, 1)
    br2c = br2.reshape(_SW, 1)

    full = lambda shape: pl.BlockSpec(shape, lambda b: (0,) * len(shape))
    y = pl.pallas_call(
        _gater_body,
        grid=(_B,),
        in_specs=[
            pl.BlockSpec((1, _SW, 1), lambda b: (b, 0, 0)),
            pl.BlockSpec((1, _SW, 1), lambda b: (b, 0, 0)),
            pl.BlockSpec((1, _C, _P), lambda b: (b, 0, 0)),
            pl.BlockSpec((1, _SW, _C, _P), lambda b: (b, 0, 0, 0)),
            full((_SW, _C, _C)),
            full((_SW, _C, 1)),
            full((_SW, _RH, _C)),
            full((_RH, _C)),
            full((_RH, 1)),
            full((_SW, _RH)),
            full((_SW, 1)),
        ],
        out_specs=pl.BlockSpec((1, _C, _P), lambda b: (b, 0, 0)),
        out_shape=jax.ShapeDtypeStruct((_B, _C, _P), jnp.float32),
        scratch_shapes=[pltpu.VMEM((_SW, _C, _P), jnp.float32)],
    )(cur2, prev2, S3, T4, W_ad, b_ad3, U_T, M_S, c0, Wr2, br2c)

    return y.reshape(_B, _C, _H, _W)


# speedup vs baseline: 3.2350x; 1.0287x over previous
import jax
import jax.numpy as jnp
from jax.experimental import pallas as pl
from jax.experimental.pallas import tpu as pltpu

_B, _SW, _C, _H, _W = 8, 8, 192, 24, 24
_P = _H * _W
_RH = 64

_TEMP = 0.7
_ALPHA_ADV, _BETA_BAND = 1.0, 0.5
_BAND_L, _BAND_H = 0.05, 0.2
_W_HEUR, _W_LEAR = 0.5, 0.5

_KPREC = jax.lax.Precision.DEFAULT


def _gater_body(cur_ref, prev_ref, S_ref, T_ref, W_ad_ref, b_ad_ref,
                Wr1_ref, br1_ref, Wr2_ref, br2_ref,
                y_ref, Tz_ref, MS_ref, c0_ref, UT_ref):
    b = pl.program_id(0)

    @pl.when(b == 0)
    def _fold_weights():
        ms = jnp.zeros((_RH, _C), jnp.float32)
        c0 = br1_ref[...]
        for s in range(_SW):
            A_S = Wr1_ref[:, s, 0]
            A_T = Wr1_ref[:, s, 1]
            A_D = Wr1_ref[:, s, 2]
            WS = A_S - A_D
            UT_ref[s] = A_T + A_D
            ms = ms + jnp.dot(WS, W_ad_ref[s], precision=_KPREC,
                              preferred_element_type=jnp.float32)
            c0 = c0 + jnp.dot(WS, b_ad_ref[s], precision=_KPREC,
                              preferred_element_type=jnp.float32)
        MS_ref[...] = ms
        c0_ref[...] = c0

    hid = jnp.dot(MS_ref[...], S_ref[0], precision=_KPREC,
                  preferred_element_type=jnp.float32)
    hid = hid + c0_ref[...]
    for s in range(_SW):
        Tz_s = jnp.dot(W_ad_ref[s], T_ref[0, s], precision=_KPREC,
                       preferred_element_type=jnp.float32)
        Tz_s = Tz_s + b_ad_ref[s]
        Tz_ref[s] = Tz_s
        hid = hid + jnp.dot(UT_ref[s], Tz_s, precision=_KPREC,
                            preferred_element_type=jnp.float32)

    h = jnp.maximum(hid, 0.0)
    learned = jnp.dot(Wr2_ref[...], h, precision=_KPREC,
                      preferred_element_type=jnp.float32) + br2_ref[...]

    cur_c = cur_ref[0]
    prev_c = prev_ref[0]
    impr = prev_c - cur_c
    adv = impr - jnp.mean(impr, axis=0, keepdims=True)
    below = jnp.maximum(_BAND_L - cur_c, 0.0)
    above = jnp.maximum(cur_c - _BAND_H, 0.0)
    band = -(below * below + above * above)
    heur = _ALPHA_ADV * adv + _BETA_BAND * band

    logits = (_W_HEUR * heur + _W_LEAR * learned) / _TEMP
    m = jnp.max(logits, axis=0, keepdims=True)
    e = jnp.exp(logits - m)
    g = e / jnp.sum(e, axis=0, keepdims=True)

    acc = g[0:1, :] * Tz_ref[0]
    for s in range(1, _SW):
        acc = acc + g[s:s + 1, :] * Tz_ref[s]
    y_ref[0] = acc


def kernel(S, T, cur, prev, W_ad, b_ad, Wr1, br1, Wr2, br2):
    S3 = S.reshape(_B, _C, _P)
    T4 = T.reshape(_B, _SW, _C, _P)
    cur2 = cur.reshape(_B, _SW, 1)
    prev2 = prev.reshape(_B, _SW, 1)
    Wr1_4 = Wr1.reshape(_RH, _SW, 3, _C)
    br1c = br1.reshape(_RH, 1)
    br2c = br2.reshape(_SW, 1)
    b_ad3 = b_ad.reshape(_SW, _C, 1)

    full = lambda shape: pl.BlockSpec(shape, lambda b: (0,) * len(shape))
    y = pl.pallas_call(
        _gater_body,
        grid=(_B,),
        in_specs=[
            pl.BlockSpec((1, _SW, 1), lambda b: (b, 0, 0)),
            pl.BlockSpec((1, _SW, 1), lambda b: (b, 0, 0)),
            pl.BlockSpec((1, _C, _P), lambda b: (b, 0, 0)),
            pl.BlockSpec((1, _SW, _C, _P), lambda b: (b, 0, 0, 0)),
            full((_SW, _C, _C)),
            full((_SW, _C, 1)),
            full((_RH, _SW, 3, _C)),
            full((_RH, 1)),
            full((_SW, _RH)),
            full((_SW, 1)),
        ],
        out_specs=pl.BlockSpec((1, _C, _P), lambda b: (b, 0, 0)),
        out_shape=jax.ShapeDtypeStruct((_B, _C, _P), jnp.float32),
        scratch_shapes=[
            pltpu.VMEM((_SW, _C, _P), jnp.float32),
            pltpu.VMEM((_RH, _C), jnp.float32),
            pltpu.VMEM((_RH, 1), jnp.float32),
            pltpu.VMEM((_SW, _RH, _C), jnp.float32),
        ],
    )(cur2, prev2, S3, T4, W_ad, b_ad3, Wr1_4, br1c, Wr2, br2c)

    return y.reshape(_B, _C, _H, _W)


# speedup vs baseline: 8.0419x; 2.4859x over previous
import jax
import jax.numpy as jnp
from jax.experimental import pallas as pl
from jax.experimental.pallas import tpu as pltpu

_B, _SW, _C, _H, _W = 8, 8, 192, 24, 24
_P = _H * _W
_RH = 64

_TEMP = 0.7
_ALPHA_ADV, _BETA_BAND = 1.0, 0.5
_BAND_L, _BAND_H = 0.05, 0.2
_W_HEUR, _W_LEAR = 0.5, 0.5

_KPREC = jax.lax.Precision.DEFAULT


def _dot_nt(a, b):
    return jax.lax.dot_general(a, b, (((1,), (1,)), ((), ())),
                               precision=_KPREC,
                               preferred_element_type=jnp.float32)


def _gater_body(cur_ref, prev_ref, S_ref, T_ref, W_ad_ref, b_ad_ref,
                Wr1_ref, br1_ref, Wr2_ref, br2_ref,
                y_ref, Tz_ref, MS_ref, c0_ref, UT_ref):
    b = pl.program_id(0)

    @pl.when(b == 0)
    def _fold_weights():
        ms = jnp.zeros((_RH, _C), jnp.float32)
        c0 = br1_ref[...]
        for s in range(_SW):
            A_S = Wr1_ref[:, s, 0]
            A_T = Wr1_ref[:, s, 1]
            A_D = Wr1_ref[:, s, 2]
            WS = A_S - A_D
            UT_ref[s] = A_T + A_D
            ms = ms + jnp.dot(WS, W_ad_ref[s], precision=_KPREC,
                              preferred_element_type=jnp.float32)
            c0 = c0 + _dot_nt(b_ad_ref[s], WS)
        MS_ref[...] = ms
        c0_ref[...] = c0

    hid = _dot_nt(S_ref[0], MS_ref[...])
    hid = hid + c0_ref[...]
    for s in range(_SW):
        Tz_s = _dot_nt(T_ref[0, s], W_ad_ref[s])
        Tz_s = Tz_s + b_ad_ref[s]
        Tz_ref[s] = Tz_s
        hid = hid + _dot_nt(Tz_s, UT_ref[s])

    h = jnp.maximum(hid, 0.0)
    learned = _dot_nt(h, Wr2_ref[...]) + br2_ref[...]

    cur_r = cur_ref[0]
    prev_r = prev_ref[0]
    impr = prev_r - cur_r
    adv = impr - jnp.mean(impr, axis=1, keepdims=True)
    below = jnp.maximum(_BAND_L - cur_r, 0.0)
    above = jnp.maximum(cur_r - _BAND_H, 0.0)
    band = -(below * below + above * above)
    heur = _ALPHA_ADV * adv + _BETA_BAND * band

    logits = (_W_HEUR * heur + _W_LEAR * learned) / _TEMP
    m = jnp.max(logits, axis=1, keepdims=True)
    e = jnp.exp(logits - m)
    g = e / jnp.sum(e, axis=1, keepdims=True)

    acc = g[:, 0:1] * Tz_ref[0]
    for s in range(1, _SW):
        acc = acc + g[:, s:s + 1] * Tz_ref[s]
    y_ref[0] = acc


def kernel(S, T, cur, prev, W_ad, b_ad, Wr1, br1, Wr2, br2):
    S3 = jnp.transpose(S, (0, 2, 3, 1)).reshape(_B, _P, _C)
    T4 = jnp.transpose(T, (0, 1, 3, 4, 2)).reshape(_B, _SW, _P, _C)
    cur2 = cur.reshape(_B, 1, _SW)
    prev2 = prev.reshape(_B, 1, _SW)
    Wr1_4 = Wr1.reshape(_RH, _SW, 3, _C)
    br1r = br1.reshape(1, _RH)
    br2r = br2.reshape(1, _SW)
    b_ad3 = b_ad.reshape(_SW, 1, _C)

    full = lambda shape: pl.BlockSpec(shape, lambda b: (0,) * len(shape))
    y = pl.pallas_call(
        _gater_body,
        grid=(_B,),
        in_specs=[
            pl.BlockSpec((1, 1, _SW), lambda b: (b, 0, 0)),
            pl.BlockSpec((1, 1, _SW), lambda b: (b, 0, 0)),
            pl.BlockSpec((1, _P, _C), lambda b: (b, 0, 0)),
            pl.BlockSpec((1, _SW, _P, _C), lambda b: (b, 0, 0, 0)),
            full((_SW, _C, _C)),
            full((_SW, 1, _C)),
            full((_RH, _SW, 3, _C)),
            full((1, _RH)),
            full((_SW, _RH)),
            full((1, _SW)),
        ],
        out_specs=pl.BlockSpec((1, _P, _C), lambda b: (b, 0, 0)),
        out_shape=jax.ShapeDtypeStruct((_B, _P, _C), jnp.float32),
        scratch_shapes=[
            pltpu.VMEM((_SW, _P, _C), jnp.float32),
            pltpu.VMEM((_RH, _C), jnp.float32),
            pltpu.VMEM((1, _RH), jnp.float32),
            pltpu.VMEM((_SW, _RH, _C), jnp.float32),
        ],
    )(cur2, prev2, S3, T4, W_ad, b_ad3, Wr1_4, br1r, Wr2, br2r)

    return jnp.transpose(y.reshape(_B, _H, _W, _C), (0, 3, 1, 2))
